# baseline (speedup 1.0000x reference)
import jax
import jax.numpy as jnp
from jax.experimental import pallas as pl
from jax.experimental.pallas import tpu as pltpu
from jax.experimental.pallas import tpu_sc as plsc

_WINDOW = 32
_LANE = 128


def kernel(input, pe_table):
    length = input.shape[1]
    max_len, d = pe_table.shape
    positions = jnp.minimum(jnp.arange(length, dtype=jnp.int32), max_len - 1)
    nblk = length // _WINDOW
    indices = jnp.zeros((nblk, _LANE), jnp.int32)
    indices = indices.at[:, :_WINDOW].set(positions.reshape(nblk, _WINDOW))

    mesh = plsc.VectorSubcoreMesh(core_axis_name="core", subcore_axis_name="subcore")

    @pl.kernel(out_type=jax.ShapeDtypeStruct((length, d), pe_table.dtype),
               mesh=mesh)
    def sc_gather_rows(pe_hbm, i_hbm, o_hbm):
        def body(i_vmem, o_vmem):
            pltpu.sync_copy(pe_hbm.at[i_vmem.at[0, pl.ds(0, _WINDOW)]], o_vmem)

        pltpu.emit_pipeline(
            body,
            grid=(nblk,),
            in_specs=[pl.BlockSpec((1, _LANE), lambda i: (i, 0))],
            out_specs=[pl.BlockSpec((_WINDOW, d), lambda i: (i, 0))],
            core_axis_name=("core", "subcore"),
            dimension_semantics=(pltpu.PARALLEL,),
        )(i_hbm, o_hbm)

    return sc_gather_rows(pe_table, indices)[None]
